# Initial kernel scaffold; baseline (speedup 1.0000x reference)
import jax
import jax.numpy as jnp
from jax.experimental import pallas as pl


def kernel(user_x, book_x, Wu, bu, Wb, bb, l1_ub_W, l1_ub_b, l1_ub_R, l1_bu_W, l1_bu_b, l1_bu_R, l2_ub_W, l2_ub_b, l2_ub_R, l2_bu_W, l2_bu_b, l2_bu_R, d1_W, d1_b, d2_W, d2_b, edge_index, edge_label_index):
    raise NotImplementedError("write your pallas kernel here")


# speedup vs baseline: 1.8591x; 1.8591x over previous
import functools

import jax
import jax.numpy as jnp
from jax import lax
from jax.experimental import pallas as pl
from jax.experimental.pallas import tpu as pltpu
from jax.experimental.pallas import tpu_sc as plsc

NU = 10000
NB = 10000
E = 160000
L = 100000
D = 256

NC = 2
NS = 16
NW = NC * NS

NP = 10240
PAD_IDX = NP - 1

EP = 163840
K_E = 128
NCH_S = (EP // NS) // K_E
NCH_W = (EP // NW) // K_E
LP = 100352
K_L = 112
NCH_L = (LP // NW) // K_L


@functools.cache
def _sc_kernels():
  mesh = plsc.VectorSubcoreMesh(core_axis_name="c", subcore_axis_name="s",
                                num_cores=NC, num_subcores=NS)

  @functools.partial(
      pl.kernel,
      out_type=jax.ShapeDtypeStruct((2, NP, 16), jnp.float32),
      mesh=mesh,
      compiler_params=pltpu.CompilerParams(use_tc_tiling_on_sc=False),
      scratch_types=[
          pltpu.VMEM((NCH_S, K_E), jnp.int32),
          pltpu.VMEM((K_E, 16), jnp.float32),
          pltpu.VMEM_SHARED((NP, 16), jnp.float32),
      ],
  )
  def deg_kernel(idx2_hbm, ones_hbm, zeros_hbm, out_hbm, si2d, ones_v, acc_sh):
    c = lax.axis_index("c")
    s = lax.axis_index("s")
    zrows = NP // NS
    pltpu.sync_copy(zeros_hbm.at[pl.ds(s * zrows, zrows)],
                    acc_sh.at[pl.ds(s * zrows, zrows)])
    pltpu.sync_copy(ones_hbm, ones_v)
    pltpu.sync_copy(idx2_hbm.at[c, s], si2d)
    plsc.subcore_barrier()

    def chunk(i, _):
      pltpu.sync_copy(ones_v, acc_sh.at[si2d.at[i]], add=True)
      return 0

    lax.fori_loop(0, NCH_S, chunk, 0)
    plsc.subcore_barrier()
    pltpu.sync_copy(acc_sh.at[pl.ds(s * zrows, zrows)],
                    out_hbm.at[c, pl.ds(s * zrows, zrows)])

  @functools.partial(
      pl.kernel,
      out_type=jax.ShapeDtypeStruct((2, NP, 16), jnp.float32),
      mesh=mesh,
      compiler_params=pltpu.CompilerParams(use_tc_tiling_on_sc=False),
      scratch_types=[
          pltpu.VMEM((NCH_W, K_E), jnp.int32),
          pltpu.VMEM((NCH_W, K_E), jnp.int32),
          pltpu.VMEM((K_E, 16), jnp.float32),
          pltpu.VMEM((K_E, 16), jnp.float32),
          pltpu.VMEM_SHARED((NP, 16), jnp.float32),
          pltpu.SemaphoreType.DMA,
          pltpu.SemaphoreType.DMA,
      ],
  )
  def agg16_kernel(table_hbm, gidx_hbm, sidx_hbm, zeros_hbm, out_hbm,
                   gi2d, si2d, rows_a, rows_b, acc_sh, sem_a, sem_b):
    c = lax.axis_index("c")
    s = lax.axis_index("s")
    zrows = NP // NS
    w = c * NS + s
    pltpu.sync_copy(zeros_hbm.at[pl.ds(s * zrows, zrows)],
                    acc_sh.at[pl.ds(s * zrows, zrows)])
    pltpu.sync_copy(gidx_hbm.at[w], gi2d)
    pltpu.sync_copy(sidx_hbm.at[w], si2d)
    plsc.subcore_barrier()

    pltpu.async_copy(table_hbm.at[gi2d.at[0]], rows_a, sem_a)

    def pair(p, _):
      i1 = 2 * p + 1
      pltpu.async_copy(table_hbm.at[gi2d.at[i1]], rows_b, sem_b)
      pltpu.make_async_copy(table_hbm.at[gi2d.at[0]], rows_a, sem_a).wait()
      pltpu.sync_copy(rows_a, acc_sh.at[si2d.at[2 * p]], add=True)
      pltpu.async_copy(table_hbm.at[gi2d.at[(i1 + 1) % NCH_W]], rows_a, sem_a)
      pltpu.make_async_copy(table_hbm.at[gi2d.at[0]], rows_b, sem_b).wait()
      pltpu.sync_copy(rows_b, acc_sh.at[si2d.at[i1]], add=True)
      return 0

    lax.fori_loop(0, NCH_W // 2, pair, 0)
    pltpu.make_async_copy(table_hbm.at[gi2d.at[0]], rows_a, sem_a).wait()
    plsc.subcore_barrier()
    pltpu.sync_copy(acc_sh.at[pl.ds(s * zrows, zrows)],
                    out_hbm.at[c, pl.ds(s * zrows, zrows)])

  @functools.partial(
      pl.kernel,
      out_type=jax.ShapeDtypeStruct((4, NP, 64), jnp.float32),
      mesh=mesh,
      compiler_params=pltpu.CompilerParams(use_tc_tiling_on_sc=False),
      scratch_types=[
          pltpu.VMEM((NCH_S, K_E), jnp.int32),
          pltpu.VMEM((NCH_S, K_E), jnp.int32),
          pltpu.VMEM((K_E, 64), jnp.float32),
          pltpu.VMEM((K_E, 64), jnp.float32),
          pltpu.VMEM_SHARED((NP, 64), jnp.float32),
          pltpu.SemaphoreType.DMA,
          pltpu.SemaphoreType.DMA,
      ],
  )
  def agg256_kernel(table4_hbm, gidx4_hbm, sidx_hbm, zeros_hbm, out_hbm,
                    ga2d, si2d, rows_a, rows_b, acc_sh, sem_a, sem_b):
    c = lax.axis_index("c")
    s = lax.axis_index("s")
    zrows = NP // NS
    pltpu.sync_copy(sidx_hbm.at[s], si2d)

    for half in range(2):
      qq = c * 2 + half
      pltpu.sync_copy(zeros_hbm.at[pl.ds(s * zrows, zrows)],
                      acc_sh.at[pl.ds(s * zrows, zrows)])
      pltpu.sync_copy(gidx4_hbm.at[qq, s], ga2d)
      plsc.subcore_barrier()

      pltpu.async_copy(table4_hbm.at[ga2d.at[0]], rows_a, sem_a)

      def pair(p, _):
        i1 = 2 * p + 1
        pltpu.async_copy(table4_hbm.at[ga2d.at[i1]], rows_b, sem_b)
        pltpu.make_async_copy(table4_hbm.at[ga2d.at[0]], rows_a, sem_a).wait()
        pltpu.sync_copy(rows_a, acc_sh.at[si2d.at[2 * p]], add=True)
        pltpu.async_copy(table4_hbm.at[ga2d.at[(i1 + 1) % NCH_S]], rows_a,
                         sem_a)
        pltpu.make_async_copy(table4_hbm.at[ga2d.at[0]], rows_b, sem_b).wait()
        pltpu.sync_copy(rows_b, acc_sh.at[si2d.at[i1]], add=True)
        return 0

      lax.fori_loop(0, NCH_S // 2, pair, 0)
      pltpu.make_async_copy(table4_hbm.at[ga2d.at[0]], rows_a, sem_a).wait()
      plsc.subcore_barrier()
      pltpu.sync_copy(acc_sh.at[pl.ds(s * zrows, zrows)],
                      out_hbm.at[qq, pl.ds(s * zrows, zrows)])
      plsc.subcore_barrier()

  @functools.partial(
      pl.kernel,
      out_type=jax.ShapeDtypeStruct((LP,), jnp.float32),
      mesh=mesh,
      compiler_params=pltpu.CompilerParams(use_tc_tiling_on_sc=False,
                                           needs_layout_passes=False),
      scratch_types=[
          pltpu.VMEM((NCH_L, K_L), jnp.int32),
          pltpu.VMEM((NCH_L, K_L), jnp.int32),
          pltpu.VMEM((K_L, D), jnp.float32),
          pltpu.VMEM((K_L, D), jnp.float32),
          pltpu.VMEM((K_L, D), jnp.float32),
          pltpu.VMEM((K_L, D), jnp.float32),
          pltpu.VMEM((K_L,), jnp.float32),
          pltpu.VMEM((K_L,), jnp.float32),
          pltpu.VMEM((D,), jnp.float32),
          pltpu.VMEM((16,), jnp.float32),
          pltpu.SemaphoreType.DMA,
          pltpu.SemaphoreType.DMA,
          pltpu.SemaphoreType.DMA,
          pltpu.SemaphoreType.DMA,
      ],
  )
  def decoder_kernel(pu_hbm, pb_hbm, row_hbm, col_hbm, w2_hbm, b2_hbm, out_hbm,
                     ri2d, ci2d, pur_a, pbr_a, pur_b, pbr_b, out_a, out_b,
                     w2_v, b2_v, sem_ua, sem_ba, sem_ub, sem_bb):
    c = lax.axis_index("c")
    s = lax.axis_index("s")
    wid = s * NC + c
    per_tile = NCH_L * K_L
    tile_base = wid * per_tile
    pltpu.sync_copy(w2_hbm, w2_v)
    pltpu.sync_copy(b2_hbm, b2_v)
    pltpu.sync_copy(row_hbm.at[wid], ri2d)
    pltpu.sync_copy(col_hbm.at[wid], ci2d)

    def issue(i, pur_v, pbr_v, sem_u, sem_b):
      pltpu.async_copy(pu_hbm.at[ri2d.at[i]], pur_v, sem_u)
      pltpu.async_copy(pb_hbm.at[ci2d.at[i]], pbr_v, sem_b)

    def waitg(pur_v, pbr_v, sem_u, sem_b):
      pltpu.make_async_copy(pu_hbm.at[ri2d.at[0]], pur_v, sem_u).wait()
      pltpu.make_async_copy(pb_hbm.at[ci2d.at[0]], pbr_v, sem_b).wait()

    def compute(i, pur_v, pbr_v, out_v):
      def group(g, _):
        erow = lax.iota(jnp.int32, 16) + g * 16
        acc = jnp.zeros((16,), jnp.float32)
        for d0 in range(0, D, 16):
          wchunk = w2_v[pl.ds(d0, 16)]
          for j in range(16):
            dvec = jnp.full((16,), d0 + j, jnp.int32)
            a = plsc.load_gather(pur_v, [erow, dvec])
            b = plsc.load_gather(pbr_v, [erow, dvec])
            acc = acc + jnp.maximum(a + b, 0.0) * wchunk[j]
        b2c = b2_v[pl.ds(0, 16)]
        plsc.store_scatter(out_v, [erow], acc + b2c[0])
        return 0

      lax.fori_loop(0, K_L // 16, group, 0)
      pltpu.sync_copy(out_v, out_hbm.at[pl.ds(tile_base + i * K_L, K_L)])

    issue(0, pur_a, pbr_a, sem_ua, sem_ba)

    def pair(p, _):
      i1 = 2 * p + 1
      issue(i1, pur_b, pbr_b, sem_ub, sem_bb)
      waitg(pur_a, pbr_a, sem_ua, sem_ba)
      compute(2 * p, pur_a, pbr_a, out_a)
      issue((i1 + 1) % NCH_L, pur_a, pbr_a, sem_ua, sem_ba)
      waitg(pur_b, pbr_b, sem_ub, sem_bb)
      compute(i1, pur_b, pbr_b, out_b)
      return 0

    lax.fori_loop(0, NCH_L // 2, pair, 0)
    waitg(pur_a, pbr_a, sem_ua, sem_ba)

  return deg_kernel, agg16_kernel, agg256_kernel, decoder_kernel


def _deg_kernel(*a):
  return _sc_kernels()[0](*a)


def _agg16_kernel(*a):
  return _sc_kernels()[1](*a)


def _agg256_kernel(*a):
  return _sc_kernels()[2](*a)


def _decoder_kernel(*a):
  return _sc_kernels()[3](*a)


_RB = 512


def _mm_bias(x, W, b):
    kd = x.shape[1]

    def body(x_ref, w_ref, b_ref, o_ref):
        o_ref[...] = (jnp.dot(x_ref[...], w_ref[...],
                              preferred_element_type=jnp.float32)
                      + b_ref[...])

    return pl.pallas_call(
        body,
        grid=(NP // _RB,),
        in_specs=[
            pl.BlockSpec((_RB, kd), lambda i: (i, 0)),
            pl.BlockSpec((kd, D), lambda i: (0, 0)),
            pl.BlockSpec((1, D), lambda i: (0, 0)),
        ],
        out_specs=pl.BlockSpec((_RB, D), lambda i: (i, 0)),
        out_shape=jax.ShapeDtypeStruct((NP, D), jnp.float32),
    )(x, W, b.reshape(1, D))


def _conv256(agg, deg, Wl, x, Rw, b, relu):
    def body(a0_ref, a1_ref, a2_ref, a3_ref, deg_ref, wl_ref, x_ref, rw_ref,
             b_ref, o_ref):
        invd = 1.0 / jnp.maximum(deg_ref[:, 0:1], 1.0)
        m = jnp.dot(x_ref[...], rw_ref[...], preferred_element_type=jnp.float32)
        for q, aq in enumerate((a0_ref, a1_ref, a2_ref, a3_ref)):
            m = m + jnp.dot(aq[...] * invd, wl_ref[64 * q:64 * (q + 1), :],
                            preferred_element_type=jnp.float32)
        out = m + b_ref[...]
        if relu:
            out = jnp.maximum(out, 0.0)
        o_ref[...] = out

    aspec = pl.BlockSpec((_RB, 64), lambda i: (i, 0))
    return pl.pallas_call(
        body,
        grid=(NP // _RB,),
        in_specs=[
            aspec, aspec, aspec, aspec,
            pl.BlockSpec((_RB, 16), lambda i: (i, 0)),
            pl.BlockSpec((D, D), lambda i: (0, 0)),
            pl.BlockSpec((_RB, D), lambda i: (i, 0)),
            pl.BlockSpec((D, D), lambda i: (0, 0)),
            pl.BlockSpec((1, D), lambda i: (0, 0)),
        ],
        out_specs=pl.BlockSpec((_RB, D), lambda i: (i, 0)),
        out_shape=jax.ShapeDtypeStruct((NP, D), jnp.float32),
    )(agg[0], agg[1], agg[2], agg[3], deg, Wl, x, Rw, b.reshape(1, D))


def _conv16(agg, deg, Wu16, bu, Wl, x, Rw, b):
    def body(aa_ref, ab_ref, deg_ref, wu_ref, bu_ref, wl_ref, x_ref, rw_ref,
             b_ref, o_ref):
        invd = 1.0 / jnp.maximum(deg_ref[:, 0:1], 1.0)
        m16 = (aa_ref[...] + ab_ref[...]) * invd
        mx = jnp.dot(m16, wu_ref[...],
                     preferred_element_type=jnp.float32) + bu_ref[...]
        out = (jnp.dot(mx, wl_ref[...], preferred_element_type=jnp.float32)
               + jnp.dot(x_ref[...], rw_ref[...],
                         preferred_element_type=jnp.float32)
               + b_ref[...])
        o_ref[...] = jnp.maximum(out, 0.0)

    return pl.pallas_call(
        body,
        grid=(NP // _RB,),
        in_specs=[
            pl.BlockSpec((_RB, 16), lambda i: (i, 0)),
            pl.BlockSpec((_RB, 16), lambda i: (i, 0)),
            pl.BlockSpec((_RB, 16), lambda i: (i, 0)),
            pl.BlockSpec((16, D), lambda i: (0, 0)),
            pl.BlockSpec((1, D), lambda i: (0, 0)),
            pl.BlockSpec((D, D), lambda i: (0, 0)),
            pl.BlockSpec((_RB, D), lambda i: (i, 0)),
            pl.BlockSpec((D, D), lambda i: (0, 0)),
            pl.BlockSpec((1, D), lambda i: (0, 0)),
        ],
        out_specs=pl.BlockSpec((_RB, D), lambda i: (i, 0)),
        out_shape=jax.ShapeDtypeStruct((NP, D), jnp.float32),
    )(agg[0], agg[1], deg, Wu16, bu.reshape(1, D), Wl, x, Rw, b.reshape(1, D))


def kernel(user_x, book_x, Wu, bu, Wb, bb,
           l1_ub_W, l1_ub_b, l1_ub_R, l1_bu_W, l1_bu_b, l1_bu_R,
           l2_ub_W, l2_ub_b, l2_ub_R, l2_bu_W, l2_bu_b, l2_bu_R,
           d1_W, d1_b, d2_W, d2_b,
           edge_index, edge_label_index):
    f32 = jnp.float32
    i32 = jnp.int32

    uxp = jnp.pad(user_x.astype(f32), ((0, NP - NU), (0, 16 - user_x.shape[1])))
    bxp = jnp.pad(book_x.astype(f32), ((0, NP - NB), (0, 0)))
    Wu16 = jnp.pad(Wu, ((0, 16 - Wu.shape[0]), (0, 0)))

    src = jnp.pad(edge_index[0].astype(i32), (0, EP - E), constant_values=PAD_IDX)
    dst = jnp.pad(edge_index[1].astype(i32), (0, EP - E), constant_values=PAD_IDX)
    row = jnp.pad(edge_label_index[0].astype(i32), (0, LP - L),
                  constant_values=PAD_IDX)
    col = jnp.pad(edge_label_index[1].astype(i32), (0, LP - L),
                  constant_values=PAD_IDX)

    q4 = jnp.arange(4, dtype=i32)[:, None]
    src4 = (src[None, :] * 4 + q4).reshape(4, NS, NCH_S, K_E)
    dst4 = (dst[None, :] * 4 + q4).reshape(4, NS, NCH_S, K_E)
    srcS = src.reshape(NS, NCH_S, K_E)
    dstS = dst.reshape(NS, NCH_S, K_E)
    srcW = src.reshape(NW, NCH_W, K_E)
    dstW = dst.reshape(NW, NCH_W, K_E)
    rowW = row.reshape(NW, NCH_L, K_L)
    colW = col.reshape(NW, NCH_L, K_L)

    zeros16 = jnp.zeros((NP, 16), f32)
    zeros64 = jnp.zeros((NP, 64), f32)
    ones16 = jnp.ones((K_E, 16), f32)

    degs = _deg_kernel(jnp.stack([dst, src]).reshape(2, NS, NCH_S, K_E),
                       ones16, zeros16)
    degB, degU = degs[0], degs[1]

    xu = _mm_bias(uxp, Wu16, bu)
    xb = _mm_bias(bxp, Wb, bb)

    aggU3 = _agg16_kernel(uxp, srcW, dstW, zeros16)
    xb1 = _conv16(aggU3, degB, Wu16, bu, l1_ub_W, xb, l1_ub_R, l1_ub_b)
    aggB1 = _agg256_kernel(xb.reshape(4 * NP, 64), dst4, srcS, zeros64)
    xu1 = _conv256(aggB1, degU, l1_bu_W, xu, l1_bu_R, l1_bu_b, relu=True)

    aggU2 = _agg256_kernel(xu1.reshape(4 * NP, 64), src4, dstS, zeros64)
    xb2 = _conv256(aggU2, degB, l2_ub_W, xb1, l2_ub_R, l2_ub_b, relu=False)
    aggB2 = _agg256_kernel(xb1.reshape(4 * NP, 64), dst4, srcS, zeros64)
    xu2 = _conv256(aggB2, degU, l2_bu_W, xu1, l2_bu_R, l2_bu_b, relu=False)

    pu = _mm_bias(xu2, d1_W[:D, :], d1_b)
    pb = _mm_bias(xb2, d1_W[D:, :], jnp.zeros((D,), f32))

    w2 = d2_W.reshape(-1).astype(f32)
    b2 = jnp.broadcast_to(d2_b.reshape(-1)[:1], (16,)).astype(f32)

    out = _decoder_kernel(pu, pb, rowW, colW, w2, b2)
    return out[:L]


# speedup vs baseline: 2.6606x; 1.4311x over previous
import functools

import jax
import jax.numpy as jnp
from jax import lax
from jax.experimental import pallas as pl
from jax.experimental.pallas import tpu as pltpu
from jax.experimental.pallas import tpu_sc as plsc

NU = 10000
NB = 10000
E = 160000
L = 100000
D = 256

NC = 2
NS = 16
NW = NC * NS

NP = 10240
PAD_IDX = NP - 1

EP = 163840
K_E = 128
NCH_S = (EP // NS) // K_E
NCH_W = (EP // NW) // K_E
LP = 100352
K_L = 112
NCH_L = (LP // NW) // K_L


@functools.cache
def _sc_kernels():
  mesh = plsc.VectorSubcoreMesh(core_axis_name="c", subcore_axis_name="s",
                                num_cores=NC, num_subcores=NS)

  @functools.partial(
      pl.kernel,
      out_type=jax.ShapeDtypeStruct((2, NP, 16), jnp.float32),
      mesh=mesh,
      compiler_params=pltpu.CompilerParams(use_tc_tiling_on_sc=False),
      scratch_types=[
          pltpu.VMEM((NCH_S, K_E), jnp.int32),
          pltpu.VMEM((K_E, 16), jnp.float32),
          pltpu.VMEM_SHARED((NP, 16), jnp.float32),
      ],
  )
  def deg_kernel(idx2_hbm, ones_hbm, zeros_hbm, out_hbm, si2d, ones_v, acc_sh):
    c = lax.axis_index("c")
    s = lax.axis_index("s")
    zrows = NP // NS
    pltpu.sync_copy(zeros_hbm.at[pl.ds(s * zrows, zrows)],
                    acc_sh.at[pl.ds(s * zrows, zrows)])
    pltpu.sync_copy(ones_hbm, ones_v)
    pltpu.sync_copy(idx2_hbm.at[c, s], si2d)
    plsc.subcore_barrier()

    def chunk(i, _):
      pltpu.sync_copy(ones_v, acc_sh.at[si2d.at[i]], add=True)
      return 0

    lax.fori_loop(0, NCH_S, chunk, 0)
    plsc.subcore_barrier()
    pltpu.sync_copy(acc_sh.at[pl.ds(s * zrows, zrows)],
                    out_hbm.at[c, pl.ds(s * zrows, zrows)])

  @functools.partial(
      pl.kernel,
      out_type=jax.ShapeDtypeStruct((2, NP, 16), jnp.float32),
      mesh=mesh,
      compiler_params=pltpu.CompilerParams(use_tc_tiling_on_sc=False),
      scratch_types=[
          pltpu.VMEM((NCH_W, K_E), jnp.int32),
          pltpu.VMEM((NCH_W, K_E), jnp.int32),
          pltpu.VMEM((K_E, 16), jnp.float32),
          pltpu.VMEM((K_E, 16), jnp.float32),
          pltpu.VMEM_SHARED((NP, 16), jnp.float32),
          pltpu.SemaphoreType.DMA,
          pltpu.SemaphoreType.DMA,
      ],
  )
  def agg16_kernel(table_hbm, gidx_hbm, sidx_hbm, zeros_hbm, out_hbm,
                   gi2d, si2d, rows_a, rows_b, acc_sh, sem_a, sem_b):
    c = lax.axis_index("c")
    s = lax.axis_index("s")
    zrows = NP // NS
    w = c * NS + s
    pltpu.sync_copy(zeros_hbm.at[pl.ds(s * zrows, zrows)],
                    acc_sh.at[pl.ds(s * zrows, zrows)])
    pltpu.sync_copy(gidx_hbm.at[w], gi2d)
    pltpu.sync_copy(sidx_hbm.at[w], si2d)
    plsc.subcore_barrier()

    pltpu.async_copy(table_hbm.at[gi2d.at[0]], rows_a, sem_a)

    def pair(p, _):
      i1 = 2 * p + 1
      pltpu.async_copy(table_hbm.at[gi2d.at[i1]], rows_b, sem_b)
      pltpu.make_async_copy(table_hbm.at[gi2d.at[0]], rows_a, sem_a).wait()
      pltpu.sync_copy(rows_a, acc_sh.at[si2d.at[2 * p]], add=True)
      pltpu.async_copy(table_hbm.at[gi2d.at[(i1 + 1) % NCH_W]], rows_a, sem_a)
      pltpu.make_async_copy(table_hbm.at[gi2d.at[0]], rows_b, sem_b).wait()
      pltpu.sync_copy(rows_b, acc_sh.at[si2d.at[i1]], add=True)
      return 0

    lax.fori_loop(0, NCH_W // 2, pair, 0)
    pltpu.make_async_copy(table_hbm.at[gi2d.at[0]], rows_a, sem_a).wait()
    plsc.subcore_barrier()
    pltpu.sync_copy(acc_sh.at[pl.ds(s * zrows, zrows)],
                    out_hbm.at[c, pl.ds(s * zrows, zrows)])

  @functools.partial(
      pl.kernel,
      out_type=jax.ShapeDtypeStruct((4, NP, 64), jnp.float32),
      mesh=mesh,
      compiler_params=pltpu.CompilerParams(use_tc_tiling_on_sc=False),
      scratch_types=[
          pltpu.VMEM((NCH_S, K_E), jnp.int32),
          pltpu.VMEM((NCH_S, K_E), jnp.int32),
          pltpu.VMEM((K_E, 64), jnp.float32),
          pltpu.VMEM((K_E, 64), jnp.float32),
          pltpu.VMEM_SHARED((NP, 64), jnp.float32),
          pltpu.SemaphoreType.DMA,
          pltpu.SemaphoreType.DMA,
      ],
  )
  def agg256_kernel(table4_hbm, gidx4_hbm, sidx_hbm, zeros_hbm, out_hbm,
                    ga2d, si2d, rows_a, rows_b, acc_sh, sem_a, sem_b):
    c = lax.axis_index("c")
    s = lax.axis_index("s")
    zrows = NP // NS
    pltpu.sync_copy(sidx_hbm.at[s], si2d)

    for half in range(2):
      qq = c * 2 + half
      pltpu.sync_copy(zeros_hbm.at[pl.ds(s * zrows, zrows)],
                      acc_sh.at[pl.ds(s * zrows, zrows)])
      pltpu.sync_copy(gidx4_hbm.at[qq, s], ga2d)
      plsc.subcore_barrier()

      pltpu.async_copy(table4_hbm.at[ga2d.at[0]], rows_a, sem_a)

      def pair(p, _):
        i1 = 2 * p + 1
        pltpu.async_copy(table4_hbm.at[ga2d.at[i1]], rows_b, sem_b)
        pltpu.make_async_copy(table4_hbm.at[ga2d.at[0]], rows_a, sem_a).wait()
        pltpu.sync_copy(rows_a, acc_sh.at[si2d.at[2 * p]], add=True)
        pltpu.async_copy(table4_hbm.at[ga2d.at[(i1 + 1) % NCH_S]], rows_a,
                         sem_a)
        pltpu.make_async_copy(table4_hbm.at[ga2d.at[0]], rows_b, sem_b).wait()
        pltpu.sync_copy(rows_b, acc_sh.at[si2d.at[i1]], add=True)
        return 0

      lax.fori_loop(0, NCH_S // 2, pair, 0)
      pltpu.make_async_copy(table4_hbm.at[ga2d.at[0]], rows_a, sem_a).wait()
      plsc.subcore_barrier()
      pltpu.sync_copy(acc_sh.at[pl.ds(s * zrows, zrows)],
                      out_hbm.at[qq, pl.ds(s * zrows, zrows)])
      plsc.subcore_barrier()

  @functools.partial(
      pl.kernel,
      out_type=jax.ShapeDtypeStruct((LP,), jnp.float32),
      mesh=mesh,
      compiler_params=pltpu.CompilerParams(use_tc_tiling_on_sc=False,
                                           needs_layout_passes=False),
      scratch_types=[
          pltpu.VMEM((NCH_L, K_L), jnp.int32),
          pltpu.VMEM((NCH_L, K_L), jnp.int32),
          pltpu.VMEM((K_L, D), jnp.float32),
          pltpu.VMEM((K_L, D), jnp.float32),
          pltpu.VMEM((K_L, D), jnp.float32),
          pltpu.VMEM((K_L, D), jnp.float32),
          pltpu.VMEM((K_L,), jnp.float32),
          pltpu.VMEM((K_L,), jnp.float32),
          pltpu.VMEM((256,), jnp.float32),
          pltpu.VMEM((D,), jnp.float32),
          pltpu.VMEM((16,), jnp.float32),
          pltpu.SemaphoreType.DMA,
          pltpu.SemaphoreType.DMA,
          pltpu.SemaphoreType.DMA,
          pltpu.SemaphoreType.DMA,
      ],
  )
  def decoder_kernel(pu_hbm, pb_hbm, row_hbm, col_hbm, w2_hbm, b2_hbm, out_hbm,
                     ri2d, ci2d, pur_a, pbr_a, pur_b, pbr_b, out_a, out_b,
                     red_v, w2_v, b2_v, sem_ua, sem_ba, sem_ub, sem_bb):
    c = lax.axis_index("c")
    s = lax.axis_index("s")
    wid = s * NC + c
    per_tile = NCH_L * K_L
    tile_base = wid * per_tile
    pltpu.sync_copy(w2_hbm, w2_v)
    pltpu.sync_copy(b2_hbm, b2_v)
    pltpu.sync_copy(row_hbm.at[wid], ri2d)
    pltpu.sync_copy(col_hbm.at[wid], ci2d)

    def issue(i, pur_v, pbr_v, sem_u, sem_b):
      pltpu.async_copy(pu_hbm.at[ri2d.at[i]], pur_v, sem_u)
      pltpu.async_copy(pb_hbm.at[ci2d.at[i]], pbr_v, sem_b)

    def waitg(pur_v, pbr_v, sem_u, sem_b):
      pltpu.make_async_copy(pu_hbm.at[ri2d.at[0]], pur_v, sem_u).wait()
      pltpu.make_async_copy(pb_hbm.at[ci2d.at[0]], pbr_v, sem_b).wait()

    def compute(i, pur_v, pbr_v, out_v):
      def group(g, _):
        erow = lax.iota(jnp.int32, 16) + g * 16
        e16 = lax.iota(jnp.int32, 16) * 16
        for j in range(16):
          e = g * 16 + j
          acc0 = jnp.zeros((16,), jnp.float32)
          acc1 = jnp.zeros((16,), jnp.float32)
          for d0 in range(0, D, 32):
            a0 = pur_v[e, pl.ds(d0, 16)]
            b0 = pbr_v[e, pl.ds(d0, 16)]
            acc0 = acc0 + jnp.maximum(a0 + b0, 0.0) * w2_v[pl.ds(d0, 16)]
            a1 = pur_v[e, pl.ds(d0 + 16, 16)]
            b1 = pbr_v[e, pl.ds(d0 + 16, 16)]
            acc1 = acc1 + jnp.maximum(a1 + b1, 0.0) * w2_v[pl.ds(d0 + 16, 16)]
          red_v[pl.ds(j * 16, 16)] = acc0 + acc1
        tot0 = jnp.zeros((16,), jnp.float32)
        tot1 = jnp.zeros((16,), jnp.float32)
        for l in range(0, 16, 2):
          tot0 = tot0 + plsc.load_gather(red_v, [e16 + l])
          tot1 = tot1 + plsc.load_gather(red_v, [e16 + (l + 1)])
        b2c = b2_v[pl.ds(0, 16)]
        plsc.store_scatter(out_v, [erow], tot0 + tot1 + b2c[0])
        return 0

      lax.fori_loop(0, K_L // 16, group, 0)
      pltpu.sync_copy(out_v, out_hbm.at[pl.ds(tile_base + i * K_L, K_L)])

    issue(0, pur_a, pbr_a, sem_ua, sem_ba)

    def pair(p, _):
      i1 = 2 * p + 1
      issue(i1, pur_b, pbr_b, sem_ub, sem_bb)
      waitg(pur_a, pbr_a, sem_ua, sem_ba)
      compute(2 * p, pur_a, pbr_a, out_a)
      issue((i1 + 1) % NCH_L, pur_a, pbr_a, sem_ua, sem_ba)
      waitg(pur_b, pbr_b, sem_ub, sem_bb)
      compute(i1, pur_b, pbr_b, out_b)
      return 0

    lax.fori_loop(0, NCH_L // 2, pair, 0)
    waitg(pur_a, pbr_a, sem_ua, sem_ba)

  return deg_kernel, agg16_kernel, agg256_kernel, decoder_kernel


def _deg_kernel(*a):
  return _sc_kernels()[0](*a)


def _agg16_kernel(*a):
  return _sc_kernels()[1](*a)


def _agg256_kernel(*a):
  return _sc_kernels()[2](*a)


def _decoder_kernel(*a):
  return _sc_kernels()[3](*a)


_RB = 512


def _mm_bias(x, W, b):
    kd = x.shape[1]

    def body(x_ref, w_ref, b_ref, o_ref):
        o_ref[...] = (jnp.dot(x_ref[...], w_ref[...],
                              preferred_element_type=jnp.float32)
                      + b_ref[...])

    return pl.pallas_call(
        body,
        grid=(NP // _RB,),
        in_specs=[
            pl.BlockSpec((_RB, kd), lambda i: (i, 0)),
            pl.BlockSpec((kd, D), lambda i: (0, 0)),
            pl.BlockSpec((1, D), lambda i: (0, 0)),
        ],
        out_specs=pl.BlockSpec((_RB, D), lambda i: (i, 0)),
        out_shape=jax.ShapeDtypeStruct((NP, D), jnp.float32),
    )(x, W, b.reshape(1, D))


def _conv256(agg, deg, Wl, x, Rw, b, relu):
    def body(a0_ref, a1_ref, a2_ref, a3_ref, deg_ref, wl_ref, x_ref, rw_ref,
             b_ref, o_ref):
        invd = 1.0 / jnp.maximum(deg_ref[:, 0:1], 1.0)
        m = jnp.dot(x_ref[...], rw_ref[...], preferred_element_type=jnp.float32)
        for q, aq in enumerate((a0_ref, a1_ref, a2_ref, a3_ref)):
            m = m + jnp.dot(aq[...] * invd, wl_ref[64 * q:64 * (q + 1), :],
                            preferred_element_type=jnp.float32)
        out = m + b_ref[...]
        if relu:
            out = jnp.maximum(out, 0.0)
        o_ref[...] = out

    aspec = pl.BlockSpec((_RB, 64), lambda i: (i, 0))
    return pl.pallas_call(
        body,
        grid=(NP // _RB,),
        in_specs=[
            aspec, aspec, aspec, aspec,
            pl.BlockSpec((_RB, 16), lambda i: (i, 0)),
            pl.BlockSpec((D, D), lambda i: (0, 0)),
            pl.BlockSpec((_RB, D), lambda i: (i, 0)),
            pl.BlockSpec((D, D), lambda i: (0, 0)),
            pl.BlockSpec((1, D), lambda i: (0, 0)),
        ],
        out_specs=pl.BlockSpec((_RB, D), lambda i: (i, 0)),
        out_shape=jax.ShapeDtypeStruct((NP, D), jnp.float32),
    )(agg[0], agg[1], agg[2], agg[3], deg, Wl, x, Rw, b.reshape(1, D))


def _conv16(agg, deg, Wu16, bu, Wl, x, Rw, b):
    def body(aa_ref, ab_ref, deg_ref, wu_ref, bu_ref, wl_ref, x_ref, rw_ref,
             b_ref, o_ref):
        invd = 1.0 / jnp.maximum(deg_ref[:, 0:1], 1.0)
        m16 = (aa_ref[...] + ab_ref[...]) * invd
        mx = jnp.dot(m16, wu_ref[...],
                     preferred_element_type=jnp.float32) + bu_ref[...]
        out = (jnp.dot(mx, wl_ref[...], preferred_element_type=jnp.float32)
               + jnp.dot(x_ref[...], rw_ref[...],
                         preferred_element_type=jnp.float32)
               + b_ref[...])
        o_ref[...] = jnp.maximum(out, 0.0)

    return pl.pallas_call(
        body,
        grid=(NP // _RB,),
        in_specs=[
            pl.BlockSpec((_RB, 16), lambda i: (i, 0)),
            pl.BlockSpec((_RB, 16), lambda i: (i, 0)),
            pl.BlockSpec((_RB, 16), lambda i: (i, 0)),
            pl.BlockSpec((16, D), lambda i: (0, 0)),
            pl.BlockSpec((1, D), lambda i: (0, 0)),
            pl.BlockSpec((D, D), lambda i: (0, 0)),
            pl.BlockSpec((_RB, D), lambda i: (i, 0)),
            pl.BlockSpec((D, D), lambda i: (0, 0)),
            pl.BlockSpec((1, D), lambda i: (0, 0)),
        ],
        out_specs=pl.BlockSpec((_RB, D), lambda i: (i, 0)),
        out_shape=jax.ShapeDtypeStruct((NP, D), jnp.float32),
    )(agg[0], agg[1], deg, Wu16, bu.reshape(1, D), Wl, x, Rw, b.reshape(1, D))


def kernel(user_x, book_x, Wu, bu, Wb, bb,
           l1_ub_W, l1_ub_b, l1_ub_R, l1_bu_W, l1_bu_b, l1_bu_R,
           l2_ub_W, l2_ub_b, l2_ub_R, l2_bu_W, l2_bu_b, l2_bu_R,
           d1_W, d1_b, d2_W, d2_b,
           edge_index, edge_label_index):
    f32 = jnp.float32
    i32 = jnp.int32

    uxp = jnp.pad(user_x.astype(f32), ((0, NP - NU), (0, 16 - user_x.shape[1])))
    bxp = jnp.pad(book_x.astype(f32), ((0, NP - NB), (0, 0)))
    Wu16 = jnp.pad(Wu, ((0, 16 - Wu.shape[0]), (0, 0)))

    src = jnp.pad(edge_index[0].astype(i32), (0, EP - E), constant_values=PAD_IDX)
    dst = jnp.pad(edge_index[1].astype(i32), (0, EP - E), constant_values=PAD_IDX)
    row = jnp.pad(edge_label_index[0].astype(i32), (0, LP - L),
                  constant_values=PAD_IDX)
    col = jnp.pad(edge_label_index[1].astype(i32), (0, LP - L),
                  constant_values=PAD_IDX)

    q4 = jnp.arange(4, dtype=i32)[:, None]
    src4 = (src[None, :] * 4 + q4).reshape(4, NS, NCH_S, K_E)
    dst4 = (dst[None, :] * 4 + q4).reshape(4, NS, NCH_S, K_E)
    srcS = src.reshape(NS, NCH_S, K_E)
    dstS = dst.reshape(NS, NCH_S, K_E)
    srcW = src.reshape(NW, NCH_W, K_E)
    dstW = dst.reshape(NW, NCH_W, K_E)
    rowW = row.reshape(NW, NCH_L, K_L)
    colW = col.reshape(NW, NCH_L, K_L)

    zeros16 = jnp.zeros((NP, 16), f32)
    zeros64 = jnp.zeros((NP, 64), f32)
    ones16 = jnp.ones((K_E, 16), f32)

    degs = _deg_kernel(jnp.stack([dst, src]).reshape(2, NS, NCH_S, K_E),
                       ones16, zeros16)
    degB, degU = degs[0], degs[1]

    xu = _mm_bias(uxp, Wu16, bu)
    xb = _mm_bias(bxp, Wb, bb)

    aggU3 = _agg16_kernel(uxp, srcW, dstW, zeros16)
    xb1 = _conv16(aggU3, degB, Wu16, bu, l1_ub_W, xb, l1_ub_R, l1_ub_b)
    aggB1 = _agg256_kernel(xb.reshape(4 * NP, 64), dst4, srcS, zeros64)
    xu1 = _conv256(aggB1, degU, l1_bu_W, xu, l1_bu_R, l1_bu_b, relu=True)

    aggU2 = _agg256_kernel(xu1.reshape(4 * NP, 64), src4, dstS, zeros64)
    xb2 = _conv256(aggU2, degB, l2_ub_W, xb1, l2_ub_R, l2_ub_b, relu=False)
    aggB2 = _agg256_kernel(xb1.reshape(4 * NP, 64), dst4, srcS, zeros64)
    xu2 = _conv256(aggB2, degU, l2_bu_W, xu1, l2_bu_R, l2_bu_b, relu=False)

    pu = _mm_bias(xu2, d1_W[:D, :], d1_b)
    pb = _mm_bias(xb2, d1_W[D:, :], jnp.zeros((D,), f32))

    w2 = d2_W.reshape(-1).astype(f32)
    b2 = jnp.broadcast_to(d2_b.reshape(-1)[:1], (16,)).astype(f32)

    out = _decoder_kernel(pu, pb, rowW, colW, w2, b2)
    return out[:L]
